# Initial kernel scaffold; baseline (speedup 1.0000x reference)
import jax
import jax.numpy as jnp
from jax.experimental import pallas as pl


def kernel(patches):
    raise NotImplementedError("write your pallas kernel here")


# speedup vs baseline: 42.7232x; 42.7232x over previous
import functools
import math

import numpy as np

import jax
import jax.numpy as jnp
from jax import lax
from jax.experimental import pallas as pl
from jax.experimental.pallas import tpu as pltpu
from jax.experimental.pallas import tpu_sc as plsc

T, B, C = 1024, 128, 768
REMAIN_T = T // 2


def _host_indexes():
    side = int(math.sqrt(T))
    rng = np.random.RandomState(0)
    rands = rng.randint(0, 2, size=B)
    fwd = np.empty((T, B), np.int32)
    bwd = np.empty((T, B), np.int32)
    base = np.arange(T).reshape(side, side)
    for b in range(B):
        if rands[b] == 0:
            fi = np.concatenate((base[0::2, :], base[1::2, :])).reshape(-1)
        else:
            fi = np.concatenate((base[1::2, :], base[0::2, :])).reshape(-1)
        fwd[:, b] = fi
        bwd[:, b] = np.argsort(fi)
    return fwd, bwd


_FWD, _BWD = _host_indexes()
_SRC = (_FWD[:REMAIN_T, :].astype(np.int64) * B
        + np.arange(B)[None, :]).astype(np.int32).reshape(-1)

_NW = 32
_ROWS = REMAIN_T * B
_RPW = _ROWS // _NW
_K = 128
_CHUNKS = _RPW // _K

_mesh = plsc.VectorSubcoreMesh(core_axis_name="c", subcore_axis_name="s")


@functools.partial(
    pl.kernel,
    mesh=_mesh,
    out_type=jax.ShapeDtypeStruct((_ROWS, C), jnp.float32),
    scratch_types=[
        pltpu.VMEM((_K,), jnp.int32),
        pltpu.VMEM((_K, C), jnp.float32),
        pltpu.SemaphoreType.DMA,
    ],
)
def _gather_rows(table_hbm, src_hbm, out_hbm, idx_v, rows_v, sem):
    wid = lax.axis_index("s") * 2 + lax.axis_index("c")
    base = wid * _RPW

    def body(i, carry):
        off = base + i * _K
        pltpu.sync_copy(src_hbm.at[pl.ds(off, _K)], idx_v)
        pltpu.async_copy(table_hbm.at[idx_v], rows_v, sem).wait()
        pltpu.sync_copy(rows_v, out_hbm.at[pl.ds(off, _K)])
        return carry

    lax.fori_loop(0, _CHUNKS, body, 0)


def kernel(patches):
    table = patches.reshape(T * B, C)
    out_flat = _gather_rows(table, jnp.asarray(_SRC))
    out = out_flat.reshape(REMAIN_T, B, C)
    return out, jnp.asarray(_FWD), jnp.asarray(_BWD)


# speedup vs baseline: 43.8286x; 1.0259x over previous
import functools
import math

import numpy as np

import jax
import jax.numpy as jnp
from jax import lax
from jax.experimental import pallas as pl
from jax.experimental.pallas import tpu as pltpu
from jax.experimental.pallas import tpu_sc as plsc

T, B, C = 1024, 128, 768
REMAIN_T = T // 2


def _host_indexes():
    side = int(math.sqrt(T))
    rng = np.random.RandomState(0)
    rands = rng.randint(0, 2, size=B)
    fwd = np.empty((T, B), np.int32)
    bwd = np.empty((T, B), np.int32)
    base = np.arange(T).reshape(side, side)
    for b in range(B):
        if rands[b] == 0:
            fi = np.concatenate((base[0::2, :], base[1::2, :])).reshape(-1)
        else:
            fi = np.concatenate((base[1::2, :], base[0::2, :])).reshape(-1)
        fwd[:, b] = fi
        bwd[:, b] = np.argsort(fi)
    return fwd, bwd


_FWD, _BWD = _host_indexes()
_SRC = (_FWD[:REMAIN_T, :].astype(np.int64) * B
        + np.arange(B)[None, :]).astype(np.int32).reshape(-1)

_NW = 32
_ROWS = REMAIN_T * B
_RPW = _ROWS // _NW
_K = 64
_CHUNKS = _RPW // _K

_mesh = plsc.VectorSubcoreMesh(core_axis_name="c", subcore_axis_name="s")


@functools.partial(
    pl.kernel,
    mesh=_mesh,
    out_type=jax.ShapeDtypeStruct((_ROWS, C), jnp.float32),
    scratch_types=[
        pltpu.VMEM((_CHUNKS, _K), jnp.int32),
        pltpu.VMEM((_K, C), jnp.float32),
        pltpu.VMEM((_K, C), jnp.float32),
        pltpu.SemaphoreType.DMA,
        pltpu.SemaphoreType.DMA,
        pltpu.SemaphoreType.DMA,
        pltpu.SemaphoreType.DMA,
    ],
)
def _gather_rows(table_hbm, src_hbm, out_hbm, idx_v, rows0, rows1, g0, g1, w0, w1):
    wid = lax.axis_index("s") * 2 + lax.axis_index("c")
    base = wid * _RPW
    bufs = (rows0, rows1)
    gsems = (g0, g1)
    wsems = (w0, w1)

    pltpu.sync_copy(src_hbm.at[wid], idx_v)

    def start_gather(c, b):
        pltpu.async_copy(table_hbm.at[idx_v.at[c]], bufs[b], gsems[b])

    def wait_gather(c, b):
        pltpu.make_async_copy(table_hbm.at[idx_v.at[c]], bufs[b], gsems[b]).wait()

    def start_write(c, b):
        pltpu.async_copy(bufs[b], out_hbm.at[pl.ds(base + c * _K, _K)], wsems[b])

    def wait_write(c, b):
        pltpu.make_async_copy(bufs[b], out_hbm.at[pl.ds(base + c * _K, _K)],
                              wsems[b]).wait()

    start_gather(0, 0)
    start_gather(1, 1)

    def body(i, carry):
        for b in range(2):
            c = 2 * i + b
            wait_gather(c, b)
            start_write(c, b)
            wait_write(c, b)
            start_gather(c + 2, b)
        return carry

    lax.fori_loop(0, (_CHUNKS - 2) // 2, body, 0)

    for b in range(2):
        c = _CHUNKS - 2 + b
        wait_gather(c, b)
        start_write(c, b)
        wait_write(c, b)


def kernel(patches):
    table = patches.reshape(T * B, C)
    out_flat = _gather_rows(table, jnp.asarray(_SRC.reshape(_NW, _CHUNKS, _K)))
    out = out_flat.reshape(REMAIN_T, B, C)
    return out, jnp.asarray(_FWD), jnp.asarray(_BWD)
